# baseline (speedup 1.0000x reference)
import functools

import jax
import jax.numpy as jnp
from jax import lax
from jax.experimental import pallas as pl
from jax.experimental.pallas import tpu as pltpu
from jax.experimental.pallas import tpu_sc as plsc

D = 128
CHUNK = 256
NBUF = 2


@functools.lru_cache(maxsize=None)
def _make_gather(B, V):
    info = plsc.get_sparse_core_info()
    NC, NS = info.num_cores, info.num_subcores
    NW = NC * NS
    assert B % (NW * CHUNK * NBUF) == 0
    b_per_w = B // NW
    n_chunks = b_per_w // CHUNK
    n_rounds = n_chunks // NBUF
    mesh = plsc.VectorSubcoreMesh(core_axis_name="c", subcore_axis_name="s")

    @functools.partial(
        pl.kernel,
        mesh=mesh,
        out_type=jax.ShapeDtypeStruct((B, D), jnp.float32),
        scratch_types=(
            [pltpu.VMEM((b_per_w,), jnp.int32),
             pltpu.VMEM((NBUF, CHUNK, D), jnp.float32)]
            + [pltpu.SemaphoreType.DMA] * (2 * NBUF)
        ),
    )
    def gather_kernel(idx_hbm, table_hbm, out_hbm, idx_v, rows_v, *sems):
        gsem, wsem = sems[:NBUF], sems[NBUF:]
        wid = lax.axis_index("s") * NC + lax.axis_index("c")
        base = wid * b_per_w
        pltpu.sync_copy(idx_hbm.at[pl.ds(base, b_per_w)], idx_v)

        def start_gather(chunk, b):
            off = pl.multiple_of(chunk * CHUNK, CHUNK)
            pltpu.async_copy(
                table_hbm.at[idx_v.at[pl.ds(off, CHUNK)]], rows_v.at[b], gsem[b]
            )

        def start_writeback(chunk, b):
            off = pl.multiple_of(chunk * CHUNK, CHUNK)
            pltpu.async_copy(
                rows_v.at[b], out_hbm.at[pl.ds(base + off, CHUNK)], wsem[b]
            )

        def wait(sem, b):
            pltpu.make_async_copy(
                out_hbm.at[pl.ds(0, CHUNK)], rows_v.at[b], sem
            ).wait()

        for b in range(NBUF):
            start_gather(b, b)

        def body(g, carry):
            for b in range(NBUF):
                wait(gsem[b], b)
                start_writeback(g * NBUF + b, b)
            for b in range(NBUF):
                wait(wsem[b], b)
                start_gather((g + 1) * NBUF + b, b)
            return carry

        lax.fori_loop(0, n_rounds - 1, body, 0)

        g_last = n_rounds - 1
        for b in range(NBUF):
            wait(gsem[b], b)
            start_writeback(g_last * NBUF + b, b)
        for b in range(NBUF):
            wait(wsem[b], b)

    return gather_kernel


def kernel(x, embeddings):
    S0, S1 = x.shape
    B = S0 * S1
    idx = x.reshape(B).astype(jnp.int32)
    out = _make_gather(B, embeddings.shape[0])(idx, embeddings)
    return out.reshape(S0, S1, D)


# speedup vs baseline: 1.0161x; 1.0161x over previous
import functools

import jax
import jax.numpy as jnp
from jax import lax
from jax.experimental import pallas as pl
from jax.experimental.pallas import tpu as pltpu
from jax.experimental.pallas import tpu_sc as plsc

D = 128
CHUNK = 200
NBUF = 4


@functools.lru_cache(maxsize=None)
def _make_gather(B, V):
    info = plsc.get_sparse_core_info()
    NC, NS = info.num_cores, info.num_subcores
    NW = NC * NS
    assert B % (NW * CHUNK * NBUF) == 0
    b_per_w = B // NW
    n_chunks = b_per_w // CHUNK
    n_rounds = n_chunks // NBUF
    mesh = plsc.VectorSubcoreMesh(core_axis_name="c", subcore_axis_name="s")

    @functools.partial(
        pl.kernel,
        mesh=mesh,
        out_type=jax.ShapeDtypeStruct((B, D), jnp.float32),
        scratch_types=(
            [pltpu.VMEM((b_per_w,), jnp.int32),
             pltpu.VMEM((NBUF, CHUNK, D), jnp.float32)]
            + [pltpu.SemaphoreType.DMA] * (2 * NBUF)
        ),
    )
    def gather_kernel(idx_hbm, table_hbm, out_hbm, idx_v, rows_v, *sems):
        gsem, wsem = sems[:NBUF], sems[NBUF:]
        wid = lax.axis_index("s") * NC + lax.axis_index("c")
        base = wid * b_per_w
        pltpu.sync_copy(idx_hbm.at[pl.ds(base, b_per_w)], idx_v)

        def start_gather(chunk, b):
            off = pl.multiple_of(chunk * CHUNK, CHUNK)
            pltpu.async_copy(
                table_hbm.at[idx_v.at[pl.ds(off, CHUNK)]], rows_v.at[b], gsem[b]
            )

        def start_writeback(chunk, b):
            off = pl.multiple_of(chunk * CHUNK, CHUNK)
            pltpu.async_copy(
                rows_v.at[b], out_hbm.at[pl.ds(base + off, CHUNK)], wsem[b]
            )

        def wait(sem, b):
            pltpu.make_async_copy(
                out_hbm.at[pl.ds(0, CHUNK)], rows_v.at[b], sem
            ).wait()

        for b in range(NBUF):
            start_gather(b, b)

        def body(g, carry):
            for b in range(NBUF):
                wait(gsem[b], b)
                start_writeback(g * NBUF + b, b)
            for b in range(NBUF):
                wait(wsem[b], b)
                start_gather((g + 1) * NBUF + b, b)
            return carry

        lax.fori_loop(0, n_rounds - 1, body, 0)

        g_last = n_rounds - 1
        for b in range(NBUF):
            wait(gsem[b], b)
            start_writeback(g_last * NBUF + b, b)
        for b in range(NBUF):
            wait(wsem[b], b)

    return gather_kernel


def kernel(x, embeddings):
    S0, S1 = x.shape
    B = S0 * S1
    idx = x.reshape(B).astype(jnp.int32)
    out = _make_gather(B, embeddings.shape[0])(idx, embeddings)
    return out.reshape(S0, S1, D)
